# baseline (speedup 1.0000x reference)
import functools
import jax
import jax.numpy as jnp
from jax import lax
from jax.experimental import pallas as pl
from jax.experimental.pallas import tpu as pltpu
from jax.experimental.pallas import tpu_sc as plsc

LANE = 16
CHUNK = 256
SUB = 128
NBUF = 3
EPS = 1e-12
_TAKE_DNUMS = lax.GatherDimensionNumbers(
    offset_dims=(), collapsed_slice_dims=(0,), start_index_map=(0,))


def _lane_take(v, idx):
    return lax.gather(v, idx[:, None], dimension_numbers=_TAKE_DNUMS,
                      slice_sizes=(1,),
                      mode=lax.GatherScatterMode.PROMISE_IN_BOUNDS)


def _vrsqrt(x):
    i = lax.bitcast_convert_type(x, jnp.int32)
    y = lax.bitcast_convert_type(
        jnp.int32(0x5F3759DF) - lax.shift_right_arithmetic(i, 1), jnp.float32)
    for _ in range(2):
        y = y * (1.5 - 0.5 * x * y * y)
    return y


def _lane_butterfly_sum(v):
    for step in (1, 2, 4, 8):
        perm = jnp.arange(LANE, dtype=jnp.int32) ^ step
        v = v + _lane_take(v, perm)
    return v


@functools.lru_cache(maxsize=None)
def _make_sc_kernel(n_rows, dim, n_pos):
    info = plsc.get_sparse_core_info()
    n_workers = info.num_cores * info.num_subcores
    assert n_rows % (n_workers * CHUNK) == 0
    rows_per_w = n_rows // n_workers
    n_chunks = rows_per_w // CHUNK
    chunks_per_w = n_chunks
    kblk = dim // LANE
    n_sub = CHUNK // SUB

    @functools.partial(
        pl.kernel,
        out_type=jax.ShapeDtypeStruct((n_rows, dim), jnp.float32),
        mesh=plsc.VectorSubcoreMesh(core_axis_name="c", subcore_axis_name="s"),
        scratch_types=[
            pltpu.VMEM((NBUF, 2, CHUNK), jnp.int32),
            pltpu.VMEM((NBUF, CHUNK, dim), jnp.float32),
            pltpu.VMEM((n_pos, dim), jnp.float32),
            pltpu.VMEM((3, dim), jnp.float32),
            pltpu.SemaphoreType.DMA,
            pltpu.SemaphoreType.DMA,
            pltpu.SemaphoreType.DMA,
        ],
    )
    def sc_kernel(ixs_hbm, tok_hbm, pos0_hbm, sdif_hbm, gam_hbm, bet_hbm,
                  out_hbm, idx_v, rows_v, pos_v, gbs_v, sem_i, sem_g, sem_o):
        wid = lax.axis_index("s") * info.num_cores + lax.axis_index("c")
        wbase = wid * rows_per_w
        wchunk = wid * chunks_per_w
        pltpu.sync_copy(pos0_hbm, pos_v)
        pltpu.sync_copy(gam_hbm, gbs_v.at[0])
        pltpu.sync_copy(bet_hbm, gbs_v.at[1])
        pltpu.sync_copy(sdif_hbm, gbs_v.at[2])
        gvec = [gbs_v[0, pl.ds(k * LANE, LANE)] for k in range(kblk)]
        bvec = [gbs_v[1, pl.ds(k * LANE, LANE)] for k in range(kblk)]
        sdif = [gbs_v[2, pl.ds(k * LANE, LANE)] for k in range(kblk)]

        def issue_idx(t, ring):
            tc = jnp.minimum(t, n_chunks - 1)
            pltpu.async_copy(ixs_hbm.at[wchunk + tc], idx_v.at[ring], sem_i)

        def drain_idx(t, ring):
            tc = jnp.minimum(t, n_chunks - 1)
            pltpu.make_async_copy(
                ixs_hbm.at[wchunk + tc], idx_v.at[ring], sem_i).wait()

        def issue_gather(ring, rb):
            for i in range(n_sub):
                pltpu.async_copy(
                    tok_hbm.at[idx_v.at[ring, 0, pl.ds(i * SUB, SUB)]],
                    rows_v.at[rb, pl.ds(i * SUB, SUB)], sem_g)

        def drain_gather(ring, rb):
            for i in range(n_sub):
                pltpu.make_async_copy(
                    tok_hbm.at[idx_v.at[ring, 0, pl.ds(i * SUB, SUB)]],
                    rows_v.at[rb, pl.ds(i * SUB, SUB)], sem_g).wait()

        def drain_out(rb, base):
            pltpu.make_async_copy(
                rows_v.at[rb], out_hbm.at[pl.ds(base, CHUNK)], sem_o).wait()

        issue_idx(0, 0)
        issue_idx(1, 1)
        drain_idx(0, 0)
        issue_gather(0, 0)

        def chunk_body(t, carry):
            b = lax.rem(t, NBUF)
            bn = lax.rem(t + 1, NBUF)
            base = wbase + t * CHUNK

            @pl.when(t >= 2)
            def _():
                drain_out(bn, wbase + (t - 2) * CHUNK)

            drain_idx(t + 1, bn)
            issue_gather(bn, bn)
            issue_idx(t + 2, lax.rem(t + 2, NBUF))

            drain_gather(b, b)

            @plsc.parallel_loop(0, CHUNK, 1, unroll=8)
            def _row(j):
                    jj = lax.rem(j, LANE)
                    seg16 = idx_v[b, 1, pl.ds((j // LANE) * LANE, LANE)]
                    p = lax.rem(base + j, n_pos)
                    s_spl = _lane_take(seg16, jnp.full((LANE,), jj, jnp.int32))
                    sf = s_spl.astype(jnp.float32)
                    e = []
                    for k in range(kblk):
                        tok_k = rows_v[b, j, pl.ds(k * LANE, LANE)]
                        pos_k = pos_v[p, pl.ds(k * LANE, LANE)]
                        e.append((tok_k + pos_k) + sf * sdif[k])
                    acc = e[0]
                    accq = e[0] * e[0]
                    for k in range(1, kblk):
                        acc = acc + e[k]
                        accq = accq + e[k] * e[k]
                    usum = _lane_butterfly_sum(acc)
                    qsum = _lane_butterfly_sum(accq)
                    u = usum * (1.0 / dim)
                    var = qsum * (1.0 / dim) - u * u
                    r = _vrsqrt(var + EPS)
                    for k in range(kblk):
                        rows_v[b, j, pl.ds(k * LANE, LANE)] = (
                            gvec[k] * ((e[k] - u) * r) + bvec[k])

            pltpu.async_copy(rows_v.at[b], out_hbm.at[pl.ds(base, CHUNK)],
                             sem_o)
            return carry

        lax.fori_loop(0, n_chunks, chunk_body, 0)

        drain_idx(n_chunks + 1, lax.rem(n_chunks + 1, NBUF))
        drain_gather(lax.rem(n_chunks, NBUF), lax.rem(n_chunks, NBUF))
        drain_out(lax.rem(n_chunks - 2, NBUF), wbase + (n_chunks - 2) * CHUNK)
        drain_out(lax.rem(n_chunks - 1, NBUF), wbase + (n_chunks - 1) * CHUNK)

    return sc_kernel


def kernel(x, seg, tok_table, pos_table, seg_table, gamma, beta):
    b, l = x.shape
    vocab, dim = tok_table.shape
    n = b * l
    nc = n // CHUNK
    ixs = jnp.stack([x.astype(jnp.int32).reshape(nc, CHUNK),
                     seg.astype(jnp.int32).reshape(nc, CHUNK)], axis=1)
    pos0 = pos_table[:l] + seg_table[0]
    sdif = seg_table[1] - seg_table[0]
    out = _make_sc_kernel(n, dim, l)(ixs, tok_table, pos0, sdif, gamma, beta)
    return out.reshape(b, l, dim)


# speedup vs baseline: 1.0344x; 1.0344x over previous
import functools
import jax
import jax.numpy as jnp
from jax import lax
from jax.experimental import pallas as pl
from jax.experimental.pallas import tpu as pltpu
from jax.experimental.pallas import tpu_sc as plsc

LANE = 16
CHUNK = 256
SUB = 128
NBUF = 3
EPS = 1e-12
_TAKE_DNUMS = lax.GatherDimensionNumbers(
    offset_dims=(), collapsed_slice_dims=(0,), start_index_map=(0,))


def _lane_take(v, idx):
    return lax.gather(v, idx[:, None], dimension_numbers=_TAKE_DNUMS,
                      slice_sizes=(1,),
                      mode=lax.GatherScatterMode.PROMISE_IN_BOUNDS)


def _vrsqrt(x):
    i = lax.bitcast_convert_type(x, jnp.int32)
    y = lax.bitcast_convert_type(
        jnp.int32(0x5F3759DF) - lax.shift_right_arithmetic(i, 1), jnp.float32)
    for _ in range(2):
        y = y * (1.5 - 0.5 * x * y * y)
    return y


def _lane_butterfly_sum(v):
    for step in (1, 2, 4, 8):
        perm = jnp.arange(LANE, dtype=jnp.int32) ^ step
        v = v + _lane_take(v, perm)
    return v


@functools.lru_cache(maxsize=None)
def _make_sc_kernel(n_rows, dim, n_pos):
    info = plsc.get_sparse_core_info()
    n_workers = info.num_cores * info.num_subcores
    assert n_rows % (n_workers * CHUNK) == 0
    rows_per_w = n_rows // n_workers
    n_chunks = rows_per_w // CHUNK
    chunks_per_w = n_chunks
    kblk = dim // LANE
    n_sub = CHUNK // SUB

    @functools.partial(
        pl.kernel,
        out_type=jax.ShapeDtypeStruct((n_rows, dim), jnp.float32),
        mesh=plsc.VectorSubcoreMesh(core_axis_name="c", subcore_axis_name="s"),
        scratch_types=[
            pltpu.VMEM((NBUF, 2, CHUNK), jnp.int32),
            pltpu.VMEM((NBUF, CHUNK, dim), jnp.float32),
            pltpu.VMEM((n_pos, dim), jnp.float32),
            pltpu.VMEM((3, dim), jnp.float32),
            pltpu.SemaphoreType.DMA,
            pltpu.SemaphoreType.DMA,
            pltpu.SemaphoreType.DMA,
        ],
    )
    def sc_kernel(ixs_hbm, tok_hbm, pos0_hbm, sdif_hbm, gam_hbm, bet_hbm,
                  out_hbm, idx_v, rows_v, pos_v, gbs_v, sem_i, sem_g, sem_o):
        wid = lax.axis_index("s") * info.num_cores + lax.axis_index("c")
        wbase = wid * rows_per_w
        wchunk = wid * chunks_per_w
        pltpu.sync_copy(pos0_hbm, pos_v)
        pltpu.sync_copy(gam_hbm, gbs_v.at[0])
        pltpu.sync_copy(bet_hbm, gbs_v.at[1])
        pltpu.sync_copy(sdif_hbm, gbs_v.at[2])
        gvec = [gbs_v[0, pl.ds(k * LANE, LANE)] for k in range(kblk)]
        bvec = [gbs_v[1, pl.ds(k * LANE, LANE)] for k in range(kblk)]
        sdif = [gbs_v[2, pl.ds(k * LANE, LANE)] for k in range(kblk)]

        def issue_idx(t, ring):
            tc = jnp.minimum(t, n_chunks - 1)
            pltpu.async_copy(ixs_hbm.at[wchunk + tc], idx_v.at[ring], sem_i)

        def drain_idx(t, ring):
            tc = jnp.minimum(t, n_chunks - 1)
            pltpu.make_async_copy(
                ixs_hbm.at[wchunk + tc], idx_v.at[ring], sem_i).wait()

        def issue_gather(ring, rb):
            for i in range(n_sub):
                pltpu.async_copy(
                    tok_hbm.at[idx_v.at[ring, 0, pl.ds(i * SUB, SUB)]],
                    rows_v.at[rb, pl.ds(i * SUB, SUB)], sem_g)

        def drain_gather(ring, rb):
            for i in range(n_sub):
                pltpu.make_async_copy(
                    tok_hbm.at[idx_v.at[ring, 0, pl.ds(i * SUB, SUB)]],
                    rows_v.at[rb, pl.ds(i * SUB, SUB)], sem_g).wait()

        def drain_out(rb, base):
            pltpu.make_async_copy(
                rows_v.at[rb], out_hbm.at[pl.ds(base, CHUNK)], sem_o).wait()

        issue_idx(0, 0)
        issue_idx(1, 1)
        drain_idx(0, 0)
        issue_gather(0, 0)

        def chunk_body(t, carry):
            b = lax.rem(t, NBUF)
            bn = lax.rem(t + 1, NBUF)
            base = wbase + t * CHUNK

            @pl.when(t >= 2)
            def _():
                drain_out(bn, wbase + (t - 2) * CHUNK)

            drain_idx(t + 1, bn)
            issue_gather(bn, bn)
            issue_idx(t + 2, lax.rem(t + 2, NBUF))

            drain_gather(b, b)

            def group_body(g, c1):
                seg16 = idx_v[b, 1, pl.ds(g * LANE, LANE)]
                pbase = base + g * LANE

                @plsc.parallel_loop(0, LANE, 1, unroll=10)
                def _row(jj):
                    j = g * LANE + jj
                    p = lax.rem(pbase + jj, n_pos)
                    s_spl = _lane_take(seg16, jnp.full((LANE,), jj, jnp.int32))
                    sf = s_spl.astype(jnp.float32)
                    e = []
                    for k in range(kblk):
                        tok_k = rows_v[b, j, pl.ds(k * LANE, LANE)]
                        pos_k = pos_v[p, pl.ds(k * LANE, LANE)]
                        e.append((tok_k + pos_k) + sf * sdif[k])
                    acc = e[0]
                    accq = e[0] * e[0]
                    for k in range(1, kblk):
                        acc = acc + e[k]
                        accq = accq + e[k] * e[k]
                    usum = _lane_butterfly_sum(acc)
                    qsum = _lane_butterfly_sum(accq)
                    u = usum * (1.0 / dim)
                    var = qsum * (1.0 / dim) - u * u
                    r = _vrsqrt(var + EPS)
                    for k in range(kblk):
                        rows_v[b, j, pl.ds(k * LANE, LANE)] = (
                            gvec[k] * ((e[k] - u) * r) + bvec[k])

                return c1

            lax.fori_loop(0, CHUNK // LANE, group_body, 0)

            pltpu.async_copy(rows_v.at[b], out_hbm.at[pl.ds(base, CHUNK)],
                             sem_o)
            return carry

        lax.fori_loop(0, n_chunks, chunk_body, 0)

        drain_idx(n_chunks + 1, lax.rem(n_chunks + 1, NBUF))
        drain_gather(lax.rem(n_chunks, NBUF), lax.rem(n_chunks, NBUF))
        drain_out(lax.rem(n_chunks - 2, NBUF), wbase + (n_chunks - 2) * CHUNK)
        drain_out(lax.rem(n_chunks - 1, NBUF), wbase + (n_chunks - 1) * CHUNK)

    return sc_kernel


def kernel(x, seg, tok_table, pos_table, seg_table, gamma, beta):
    b, l = x.shape
    vocab, dim = tok_table.shape
    n = b * l
    nc = n // CHUNK
    ixs = jnp.stack([x.astype(jnp.int32).reshape(nc, CHUNK),
                     seg.astype(jnp.int32).reshape(nc, CHUNK)], axis=1)
    pos0 = pos_table[:l] + seg_table[0]
    sdif = seg_table[1] - seg_table[0]
    out = _make_sc_kernel(n, dim, l)(ixs, tok_table, pos0, sdif, gamma, beta)
    return out.reshape(b, l, dim)


# speedup vs baseline: 1.1810x; 1.1417x over previous
import functools
import jax
import jax.numpy as jnp
from jax import lax
from jax.experimental import pallas as pl
from jax.experimental.pallas import tpu as pltpu
from jax.experimental.pallas import tpu_sc as plsc

LANE = 16
CHUNK = 256
SUB = 128
NBUF = 3
EPS = 1e-12
_TAKE_DNUMS = lax.GatherDimensionNumbers(
    offset_dims=(), collapsed_slice_dims=(0,), start_index_map=(0,))


def _lane_take(v, idx):
    return lax.gather(v, idx[:, None], dimension_numbers=_TAKE_DNUMS,
                      slice_sizes=(1,),
                      mode=lax.GatherScatterMode.PROMISE_IN_BOUNDS)


def _vrsqrt(x):
    i = lax.bitcast_convert_type(x, jnp.int32)
    y = lax.bitcast_convert_type(
        jnp.int32(0x5F3759DF) - lax.shift_right_arithmetic(i, 1), jnp.float32)
    for _ in range(2):
        y = y * (1.5 - 0.5 * x * y * y)
    return y


def _lane_butterfly_sum(v):
    for step in (1, 2, 4, 8):
        perm = jnp.arange(LANE, dtype=jnp.int32) ^ step
        v = v + _lane_take(v, perm)
    return v


@functools.lru_cache(maxsize=None)
def _make_sc_kernel(n_rows, dim, n_pos):
    info = plsc.get_sparse_core_info()
    n_workers = info.num_cores * info.num_subcores
    assert n_rows % (n_workers * CHUNK) == 0
    rows_per_w = n_rows // n_workers
    n_chunks = rows_per_w // CHUNK
    chunks_per_w = n_chunks
    kblk = dim // LANE
    n_sub = CHUNK // SUB

    @functools.partial(
        pl.kernel,
        out_type=jax.ShapeDtypeStruct((n_rows, dim), jnp.float32),
        mesh=plsc.VectorSubcoreMesh(core_axis_name="c", subcore_axis_name="s"),
        scratch_types=[
            pltpu.VMEM((NBUF, 2, CHUNK), jnp.int32),
            pltpu.VMEM((NBUF, CHUNK, dim), jnp.float32),
            pltpu.VMEM((n_pos, dim), jnp.float32),
            pltpu.VMEM((3, dim), jnp.float32),
            pltpu.SemaphoreType.DMA,
            pltpu.SemaphoreType.DMA,
            pltpu.SemaphoreType.DMA,
        ],
    )
    def sc_kernel(ixs_hbm, tok_hbm, pos0_hbm, sdif_hbm, gam_hbm, bet_hbm,
                  out_hbm, idx_v, rows_v, pos_v, gbs_v, sem_i, sem_g, sem_o):
        wid = lax.axis_index("s") * info.num_cores + lax.axis_index("c")
        wbase = wid * rows_per_w
        wchunk = wid * chunks_per_w
        pltpu.sync_copy(pos0_hbm, pos_v)
        pltpu.sync_copy(gam_hbm, gbs_v.at[0])
        pltpu.sync_copy(bet_hbm, gbs_v.at[1])
        pltpu.sync_copy(sdif_hbm, gbs_v.at[2])
        gvec = [gbs_v[0, pl.ds(k * LANE, LANE)] for k in range(kblk)]
        bvec = [gbs_v[1, pl.ds(k * LANE, LANE)] for k in range(kblk)]
        sdif = [gbs_v[2, pl.ds(k * LANE, LANE)] for k in range(kblk)]

        def issue_idx(t, ring):
            tc = jnp.minimum(t, n_chunks - 1)
            pltpu.async_copy(ixs_hbm.at[wchunk + tc], idx_v.at[ring], sem_i)

        def drain_idx(t, ring):
            tc = jnp.minimum(t, n_chunks - 1)
            pltpu.make_async_copy(
                ixs_hbm.at[wchunk + tc], idx_v.at[ring], sem_i).wait()

        def issue_gather(ring, rb):
            for i in range(n_sub):
                pltpu.async_copy(
                    tok_hbm.at[idx_v.at[ring, 0, pl.ds(i * SUB, SUB)]],
                    rows_v.at[rb, pl.ds(i * SUB, SUB)], sem_g)

        def drain_gather(ring, rb):
            for i in range(n_sub):
                pltpu.make_async_copy(
                    tok_hbm.at[idx_v.at[ring, 0, pl.ds(i * SUB, SUB)]],
                    rows_v.at[rb, pl.ds(i * SUB, SUB)], sem_g).wait()

        def drain_out(rb, base):
            pltpu.make_async_copy(
                rows_v.at[rb], out_hbm.at[pl.ds(base, CHUNK)], sem_o).wait()

        issue_idx(0, 0)
        issue_idx(1, 1)
        drain_idx(0, 0)
        issue_gather(0, 0)

        def chunk_body(t, carry):
            b = lax.rem(t, NBUF)
            bn = lax.rem(t + 1, NBUF)
            base = wbase + t * CHUNK

            @pl.when(t >= 2)
            def _():
                drain_out(bn, wbase + (t - 2) * CHUNK)

            drain_idx(t + 1, bn)
            issue_gather(bn, bn)
            issue_idx(t + 2, lax.rem(t + 2, NBUF))

            drain_gather(b, b)

            def group_body(g, c1):
                seg16 = idx_v[b, 1, pl.ds(g * LANE, LANE)]
                pbase = base + g * LANE

                @plsc.parallel_loop(0, LANE, 1, unroll=8)
                def _row(jj):
                    j = g * LANE + jj
                    p = lax.rem(pbase + jj, n_pos)
                    s_spl = _lane_take(seg16, jnp.full((LANE,), jj, jnp.int32))
                    sf = s_spl.astype(jnp.float32)
                    e = []
                    for k in range(kblk):
                        tok_k = rows_v[b, j, pl.ds(k * LANE, LANE)]
                        pos_k = pos_v[p, pl.ds(k * LANE, LANE)]
                        e.append((tok_k + pos_k) + sf * sdif[k])
                    acc = e[0]
                    accq = e[0] * e[0]
                    for k in range(1, kblk):
                        acc = acc + e[k]
                        accq = accq + e[k] * e[k]
                    usum = _lane_butterfly_sum(acc)
                    qsum = _lane_butterfly_sum(accq)
                    u = usum * (1.0 / dim)
                    var = qsum * (1.0 / dim) - u * u
                    r = _vrsqrt(var + EPS)
                    for k in range(kblk):
                        rows_v[b, j, pl.ds(k * LANE, LANE)] = (
                            gvec[k] * ((e[k] - u) * r) + bvec[k])

                return c1

            lax.fori_loop(0, CHUNK // LANE, group_body, 0)

            pltpu.async_copy(rows_v.at[b], out_hbm.at[pl.ds(base, CHUNK)],
                             sem_o)
            return carry

        lax.fori_loop(0, n_chunks, chunk_body, 0)

        drain_idx(n_chunks + 1, lax.rem(n_chunks + 1, NBUF))
        drain_gather(lax.rem(n_chunks, NBUF), lax.rem(n_chunks, NBUF))
        drain_out(lax.rem(n_chunks - 2, NBUF), wbase + (n_chunks - 2) * CHUNK)
        drain_out(lax.rem(n_chunks - 1, NBUF), wbase + (n_chunks - 1) * CHUNK)

    return sc_kernel


def kernel(x, seg, tok_table, pos_table, seg_table, gamma, beta):
    b, l = x.shape
    vocab, dim = tok_table.shape
    n = b * l
    nc = n // CHUNK
    ixs = jnp.stack([x.astype(jnp.int32).reshape(nc, CHUNK),
                     seg.astype(jnp.int32).reshape(nc, CHUNK)], axis=1)
    pos0 = pos_table[:l] + seg_table[0]
    sdif = seg_table[1] - seg_table[0]
    out = _make_sc_kernel(n, dim, l)(ixs, tok_table, pos0, sdif, gamma, beta)
    return out.reshape(b, l, dim)
